# Initial kernel scaffold; baseline (speedup 1.0000x reference)
import jax
import jax.numpy as jnp
from jax.experimental import pallas as pl


def kernel(x, edge_index, batch_index, W_gcn, b_gcn, W_out, b_out):
    raise NotImplementedError("write your pallas kernel here")


# speedup vs baseline: 32.5823x; 32.5823x over previous
import functools

import jax
import jax.numpy as jnp
from jax import lax
from jax.experimental import pallas as pl
from jax.experimental.pallas import tpu as pltpu
from jax.experimental.pallas import tpu_sc as plsc

N = 50000
E = 800000
D = 65
EMB = 32
G = 1024

NP = 50176
NSEG = 4
QH = NP // NSEG
QHW = QH * D
DUMP = 1280
CW = QHW + DUMP
CFLAT = NSEG * QHW

ES = E // 16
K = 2000
NCH = ES // K
SL = NP // 16
HSL = SL // 2
ZB = 3920
TW = QHW // 16
NZC = TW // ZB

BLK = 512
NBLK = NP // BLK


def _rsqrt_newton(v):
    iv = lax.bitcast_convert_type(v, jnp.int32)
    y = lax.bitcast_convert_type(jnp.int32(0x5F3759DF) - (iv >> 1),
                                 jnp.float32)
    for _ in range(3):
        y = y * (1.5 - 0.5 * v * y * y)
    return y


_sc_mesh = plsc.VectorSubcoreMesh(core_axis_name="c", subcore_axis_name="s")


@functools.partial(
    pl.kernel,
    mesh=_sc_mesh,
    compiler_params=pltpu.CompilerParams(needs_layout_passes=False),
    out_type=[
        jax.ShapeDtypeStruct((NP,), jnp.float32),
        jax.ShapeDtypeStruct((CFLAT,), jnp.float32),
    ],
    scratch_types=[
        pltpu.VMEM((NP,), jnp.float32),
        pltpu.VMEM((K,), jnp.int32),
        pltpu.VMEM((K,), jnp.int32),
        pltpu.VMEM((K,), jnp.int32),
        pltpu.VMEM((K,), jnp.float32),
        pltpu.VMEM((SL,), jnp.float32),
        pltpu.VMEM((ZB,), jnp.float32),
        pltpu.VMEM((ZB,), jnp.float32),
        pltpu.VMEM_SHARED((NP,), jnp.float32),
        pltpu.VMEM_SHARED((NP,), jnp.float32),
        pltpu.VMEM_SHARED((CW,), jnp.float32),
    ],
)
def _sc_scatter(x_hbm, src_hbm, dst_hbm, dinv_hbm, c_hbm,
                cx_v, srcb, dstb, idxb, valb, onesb, zbuf, drainb,
                deg_sh, cx_sh, C_sh):
    c = lax.axis_index("c")
    s = lax.axis_index("s")

    def fill_ones(i, _):
        onesb[pl.ds(i * 16, 16)] = jnp.full((16,), 1.0, jnp.float32)
        return 0
    lax.fori_loop(0, SL // 16, fill_ones, 0)

    def fill_zero(i, _):
        zbuf[pl.ds(i * 16, 16)] = jnp.zeros((16,), jnp.float32)
        return 0
    lax.fori_loop(0, ZB // 16, fill_zero, 0)

    pltpu.sync_copy(onesb, deg_sh.at[pl.ds(s * SL, SL)])

    plsc.subcore_barrier()

    def deg_chunk(j, _):
        off = s * ES + j * K
        pltpu.sync_copy(dst_hbm.at[pl.ds(off, K)], dstb)
        pltpu.sync_copy(onesb.at[pl.ds(0, K)], deg_sh.at[dstb], add=True)
        return 0
    lax.fori_loop(0, NCH, deg_chunk, 0)

    plsc.subcore_barrier()

    pltpu.sync_copy(deg_sh.at[pl.ds(s * SL, SL)], onesb)

    def newton(g, _):
        v = onesb[pl.ds(g * 16, 16)]
        onesb[pl.ds(g * 16, 16)] = _rsqrt_newton(v)
        return 0
    lax.fori_loop(0, SL // 16, newton, 0)

    @pl.when(c == 0)
    def _():
        pltpu.sync_copy(onesb, dinv_hbm.at[pl.ds(s * SL, SL)])

    for h in range(2):
        pltpu.sync_copy(x_hbm.at[pl.ds(s * SL + h * HSL, HSL)],
                        srcb.at[pl.ds(0, HSL)])

        def pack(g, _):
            xi = srcb[pl.ds(g * 16, 16)]
            dv = onesb[pl.ds(h * HSL + g * 16, 16)]
            onesb[pl.ds(h * HSL + g * 16, 16)] = (
                2.0 * xi.astype(jnp.float32) + dv)
            return 0
        lax.fori_loop(0, HSL // 16, pack, 0)

    pltpu.sync_copy(onesb, cx_sh.at[pl.ds(s * SL, SL)])

    plsc.subcore_barrier()

    pltpu.sync_copy(cx_sh, cx_v)

    def one_pass(p, _):
        q = 2 * p + c
        lo = q * QH

        def zero_c(k, _):
            pltpu.sync_copy(zbuf, C_sh.at[pl.ds(s * TW + k * ZB, ZB)])
            return 0
        lax.fori_loop(0, NZC, zero_c, 0)

        @pl.when(s == 0)
        def _():
            pltpu.sync_copy(zbuf.at[pl.ds(0, DUMP)],
                            C_sh.at[pl.ds(QHW, DUMP)])

        plsc.subcore_barrier()

        def edge_chunk(j, _):
            off = s * ES + j * K
            pltpu.sync_copy(src_hbm.at[pl.ds(off, K)], srcb)
            pltpu.sync_copy(dst_hbm.at[pl.ds(off, K)], dstb)

            def group(g, _):
                sv = srcb[pl.ds(g * 16, 16)]
                dvec = dstb[pl.ds(g * 16, 16)]
                v = plsc.load_gather(cx_v, [sv])
                xi = (v * 0.5).astype(jnp.int32)
                dval = v - 2.0 * xi.astype(jnp.float32)
                loc = dvec - lo
                inh = (loc >= 0) & (loc < QH)
                dmp = QHW + (dvec & 1023)
                idxb[pl.ds(g * 16, 16)] = jnp.where(inh, loc * D + xi, dmp)
                valb[pl.ds(g * 16, 16)] = dval
                return 0
            lax.fori_loop(0, K // 16, group, 0)

            pltpu.sync_copy(valb, C_sh.at[idxb], add=True)
            return 0
        lax.fori_loop(0, NCH, edge_chunk, 0)

        plsc.subcore_barrier()

        def drain(k, _):
            loc_off = s * TW + k * ZB
            pltpu.sync_copy(C_sh.at[pl.ds(loc_off, ZB)], drainb)
            pltpu.sync_copy(drainb, c_hbm.at[pl.ds(q * QHW + loc_off, ZB)])
            return 0
        lax.fori_loop(0, NZC, drain, 0)

        plsc.subcore_barrier()
        return 0

    lax.fori_loop(0, NSEG // 2, one_pass, 0)


def _tc_body(c_ref, dv_ref, x_ref, b_ref, wg_ref, bg_ref, wo_ref, out_ref,
             acc_ref):
    i = pl.program_id(0)
    cb = c_ref[...]
    dv = dv_ref[...]
    xb = x_ref[...]
    bb = b_ref[...]
    wgb = wg_ref[...].astype(jnp.bfloat16).astype(jnp.float32)
    ohx = jnp.where(
        lax.broadcasted_iota(jnp.int32, (BLK, D), 1) == xb, 1.0, 0.0)
    m = cb + dv * ohx
    p = lax.dot_general(m, wgb, (((1,), (0,)), ((), ())),
                        precision=lax.Precision.HIGHEST,
                        preferred_element_type=jnp.float32)
    h = jnp.tanh(dv * p + bg_ref[...])
    rows = lax.broadcasted_iota(jnp.int32, (BLK, 1), 0) + i * BLK
    h = jnp.where(rows < N, h, 0.0)
    ohb = jnp.where(
        lax.broadcasted_iota(jnp.int32, (BLK, G), 1) == bb, 1.0, 0.0)
    contrib = lax.dot_general(ohb, h, (((0,), (0,)), ((), ())),
                              precision=lax.Precision.HIGHEST,
                              preferred_element_type=jnp.float32)

    @pl.when(i == 0)
    def _():
        acc_ref[...] = jnp.zeros((G, EMB), jnp.float32)

    acc_ref[...] += contrib

    @pl.when(i == NBLK - 1)
    def _():
        pb = acc_ref[...].astype(jnp.bfloat16)
        wob = wo_ref[...].astype(jnp.bfloat16)
        out_ref[...] = lax.dot_general(
            wob, pb, (((0,), (1,)), ((), ())),
            preferred_element_type=jnp.float32)


_tc_call = pl.pallas_call(
    _tc_body,
    grid=(NBLK,),
    in_specs=[
        pl.BlockSpec((BLK, D), lambda i: (i, 0)),
        pl.BlockSpec((BLK, 1), lambda i: (i, 0)),
        pl.BlockSpec((BLK, 1), lambda i: (i, 0)),
        pl.BlockSpec((BLK, 1), lambda i: (i, 0)),
        pl.BlockSpec((D, EMB), lambda i: (0, 0)),
        pl.BlockSpec((1, EMB), lambda i: (0, 0)),
        pl.BlockSpec((EMB, 1), lambda i: (0, 0)),
    ],
    out_specs=pl.BlockSpec((1, G), lambda i: (0, 0)),
    out_shape=jax.ShapeDtypeStruct((1, G), jnp.float32),
    scratch_shapes=[pltpu.VMEM((G, EMB), jnp.float32)],
)


def kernel(x, edge_index, batch_index, W_gcn, b_gcn, W_out, b_out):
    src = edge_index[0]
    dst = edge_index[1]
    pad = NP - N
    xpad = jnp.concatenate([x, jnp.zeros((pad,), jnp.int32)])
    dinv, cflat = _sc_scatter(xpad, src, dst)
    c2 = cflat.reshape(NP, D)
    b2 = jnp.concatenate(
        [batch_index, jnp.zeros((pad,), jnp.int32)]).reshape(NP, 1)
    acc = _tc_call(c2, dinv.reshape(NP, 1), xpad.reshape(NP, 1), b2,
                   W_gcn, b_gcn.reshape(1, EMB), W_out)
    return acc.reshape(G, 1) + b_out


# speedup vs baseline: 43.1553x; 1.3245x over previous
import functools

import jax
import jax.numpy as jnp
from jax import lax
from jax.experimental import pallas as pl
from jax.experimental.pallas import tpu as pltpu
from jax.experimental.pallas import tpu_sc as plsc

N = 50000
E = 800000
D = 65
EMB = 32
G = 1024

NP = 50176
NSEG = 4
QH = NP // NSEG
QHW = QH * D
DUMP = 1280
CW = QHW + DUMP
CFLAT = NSEG * QHW

ES = E // 16
K = 2000
NCH = ES // K
SL = NP // 16
HSL = SL // 2
ZB = 3920
TW = QHW // 16
NZC = TW // ZB

BLK = 512
NBLK = NP // BLK


def _rsqrt_newton(v):
    iv = lax.bitcast_convert_type(v, jnp.int32)
    y = lax.bitcast_convert_type(jnp.int32(0x5F3759DF) - (iv >> 1),
                                 jnp.float32)
    for _ in range(3):
        y = y * (1.5 - 0.5 * v * y * y)
    return y


_sc_mesh = plsc.VectorSubcoreMesh(core_axis_name="c", subcore_axis_name="s")


@functools.partial(
    pl.kernel,
    mesh=_sc_mesh,
    compiler_params=pltpu.CompilerParams(needs_layout_passes=False),
    out_type=[
        jax.ShapeDtypeStruct((NP,), jnp.float32),
        jax.ShapeDtypeStruct((CFLAT,), jnp.float32),
    ],
    scratch_types=[
        pltpu.VMEM((NP,), jnp.float32),
        pltpu.VMEM((K,), jnp.int32),
        pltpu.VMEM((K,), jnp.int32),
        pltpu.VMEM((K,), jnp.int32),
        pltpu.VMEM((K,), jnp.float32),
        pltpu.VMEM((SL,), jnp.float32),
        pltpu.VMEM((ZB,), jnp.float32),
        pltpu.VMEM((ZB,), jnp.float32),
        pltpu.VMEM_SHARED((NP,), jnp.float32),
        pltpu.VMEM_SHARED((NP,), jnp.float32),
        pltpu.VMEM_SHARED((CW,), jnp.float32),
    ],
)
def _sc_scatter(x_hbm, src_hbm, dst_hbm, dinv_hbm, c_hbm,
                cx_v, srcb, dstb, idxb, valb, onesb, zbuf, drainb,
                deg_sh, cx_sh, C_sh):
    c = lax.axis_index("c")
    s = lax.axis_index("s")

    def fill_ones(i, _):
        onesb[pl.ds(i * 16, 16)] = jnp.full((16,), 1.0, jnp.float32)
        return 0
    lax.fori_loop(0, SL // 16, fill_ones, 0)

    def fill_zero(i, _):
        zbuf[pl.ds(i * 16, 16)] = jnp.zeros((16,), jnp.float32)
        return 0
    lax.fori_loop(0, ZB // 16, fill_zero, 0)

    pltpu.sync_copy(onesb, deg_sh.at[pl.ds(s * SL, SL)])

    plsc.subcore_barrier()

    def deg_chunk(j, _):
        off = s * ES + j * K
        pltpu.sync_copy(dst_hbm.at[pl.ds(off, K)], dstb)
        pltpu.sync_copy(onesb.at[pl.ds(0, K)], deg_sh.at[dstb], add=True)
        return 0
    lax.fori_loop(0, NCH, deg_chunk, 0)

    plsc.subcore_barrier()

    pltpu.sync_copy(deg_sh.at[pl.ds(s * SL, SL)], onesb)

    def newton(g, _):
        v = onesb[pl.ds(g * 16, 16)]
        onesb[pl.ds(g * 16, 16)] = _rsqrt_newton(v)
        return 0
    lax.fori_loop(0, SL // 16, newton, 0)

    @pl.when(c == 0)
    def _():
        pltpu.sync_copy(onesb, dinv_hbm.at[pl.ds(s * SL, SL)])

    for h in range(2):
        pltpu.sync_copy(x_hbm.at[pl.ds(s * SL + h * HSL, HSL)],
                        srcb.at[pl.ds(0, HSL)])

        def pack(g, _):
            xi = srcb[pl.ds(g * 16, 16)]
            dv = onesb[pl.ds(h * HSL + g * 16, 16)]
            onesb[pl.ds(h * HSL + g * 16, 16)] = (
                2.0 * xi.astype(jnp.float32) + dv)
            return 0
        lax.fori_loop(0, HSL // 16, pack, 0)

    pltpu.sync_copy(onesb, cx_sh.at[pl.ds(s * SL, SL)])

    plsc.subcore_barrier()

    pltpu.sync_copy(cx_sh, cx_v)

    def one_pass(p, _):
        q = 2 * p + c
        lo = q * QH

        def zero_c(k, _):
            pltpu.sync_copy(zbuf, C_sh.at[pl.ds(s * TW + k * ZB, ZB)])
            return 0
        lax.fori_loop(0, NZC, zero_c, 0)

        @pl.when(s == 0)
        def _():
            pltpu.sync_copy(zbuf.at[pl.ds(0, DUMP)],
                            C_sh.at[pl.ds(QHW, DUMP)])

        plsc.subcore_barrier()

        def edge_chunk(j, _):
            off = s * ES + j * K
            pltpu.sync_copy(src_hbm.at[pl.ds(off, K)], srcb)
            pltpu.sync_copy(dst_hbm.at[pl.ds(off, K)], dstb)

            def group(g, _):
                sv = srcb[pl.ds(g * 16, 16)]
                dvec = dstb[pl.ds(g * 16, 16)]
                v = plsc.load_gather(cx_v, [sv])
                xi = (v * 0.5).astype(jnp.int32)
                dval = v - 2.0 * xi.astype(jnp.float32)
                loc = dvec - lo
                inh = (loc >= 0) & (loc < QH)
                dmp = QHW + (dvec & 1023)
                idxb[pl.ds(g * 16, 16)] = jnp.where(inh, loc * D + xi, dmp)
                valb[pl.ds(g * 16, 16)] = dval
                return 0
            lax.fori_loop(0, K // 16, group, 0)

            pltpu.sync_copy(valb, C_sh.at[idxb], add=True)
            return 0
        lax.fori_loop(0, NCH, edge_chunk, 0)

        plsc.subcore_barrier()

        def drain(k, _):
            loc_off = s * TW + k * ZB
            pltpu.sync_copy(C_sh.at[pl.ds(loc_off, ZB)], drainb)
            pltpu.sync_copy(drainb, c_hbm.at[pl.ds(q * QHW + loc_off, ZB)])
            return 0
        lax.fori_loop(0, NZC, drain, 0)

        plsc.subcore_barrier()
        return 0

    lax.fori_loop(0, NSEG // 2, one_pass, 0)


def _tc_body(c_ref, dv_ref, x_ref, b_ref, wg_ref, bg_ref, wo_ref, out_ref,
             acc_ref):
    i = pl.program_id(0)
    cb = c_ref[...]
    dv = dv_ref[...]
    xb = x_ref[...]
    bb = b_ref[...]
    wgb = wg_ref[...].astype(jnp.bfloat16)
    ohx = jnp.where(
        lax.broadcasted_iota(jnp.int32, (BLK, D), 1) == xb, 1.0, 0.0)
    m = cb + dv * ohx
    mb = m.astype(jnp.bfloat16)
    mrb = (m - mb.astype(jnp.float32)).astype(jnp.bfloat16)
    dn = (((1,), (0,)), ((), ()))
    p = (lax.dot_general(mb, wgb, dn, preferred_element_type=jnp.float32)
         + lax.dot_general(mrb, wgb, dn, preferred_element_type=jnp.float32))
    h = jnp.tanh(dv * p + bg_ref[...])
    rows = lax.broadcasted_iota(jnp.int32, (BLK, 1), 0) + i * BLK
    h = jnp.where(rows < N, h, 0.0)
    ohb = (lax.broadcasted_iota(jnp.int32, (BLK, G), 1) == bb).astype(
        jnp.bfloat16)
    hb = h.astype(jnp.bfloat16)
    hrb = (h - hb.astype(jnp.float32)).astype(jnp.bfloat16)
    dp = (((0,), (0,)), ((), ()))
    contrib = (lax.dot_general(ohb, hb, dp,
                               preferred_element_type=jnp.float32)
               + lax.dot_general(ohb, hrb, dp,
                                 preferred_element_type=jnp.float32))

    @pl.when(i == 0)
    def _():
        acc_ref[...] = jnp.zeros((G, EMB), jnp.float32)

    acc_ref[...] += contrib

    @pl.when(i == NBLK - 1)
    def _():
        pb = acc_ref[...].astype(jnp.bfloat16)
        wob = wo_ref[...].astype(jnp.bfloat16)
        out_ref[...] = lax.dot_general(
            wob, pb, (((0,), (1,)), ((), ())),
            preferred_element_type=jnp.float32)


_tc_call = pl.pallas_call(
    _tc_body,
    grid=(NBLK,),
    in_specs=[
        pl.BlockSpec((BLK, D), lambda i: (i, 0)),
        pl.BlockSpec((BLK, 1), lambda i: (i, 0)),
        pl.BlockSpec((BLK, 1), lambda i: (i, 0)),
        pl.BlockSpec((BLK, 1), lambda i: (i, 0)),
        pl.BlockSpec((D, EMB), lambda i: (0, 0)),
        pl.BlockSpec((1, EMB), lambda i: (0, 0)),
        pl.BlockSpec((EMB, 1), lambda i: (0, 0)),
    ],
    out_specs=pl.BlockSpec((1, G), lambda i: (0, 0)),
    out_shape=jax.ShapeDtypeStruct((1, G), jnp.float32),
    scratch_shapes=[pltpu.VMEM((G, EMB), jnp.float32)],
)


def kernel(x, edge_index, batch_index, W_gcn, b_gcn, W_out, b_out):
    src = edge_index[0]
    dst = edge_index[1]
    pad = NP - N
    xpad = jnp.concatenate([x, jnp.zeros((pad,), jnp.int32)])
    dinv, cflat = _sc_scatter(xpad, src, dst)
    c2 = cflat.reshape(NP, D)
    b2 = jnp.concatenate(
        [batch_index, jnp.zeros((pad,), jnp.int32)]).reshape(NP, 1)
    acc = _tc_call(c2, dinv.reshape(NP, 1), xpad.reshape(NP, 1), b2,
                   W_gcn, b_gcn.reshape(1, EMB), W_out)
    return acc.reshape(G, 1) + b_out


# speedup vs baseline: 53.2163x; 1.2331x over previous
import functools

import jax
import jax.numpy as jnp
from jax import lax
from jax.experimental import pallas as pl
from jax.experimental.pallas import tpu as pltpu
from jax.experimental.pallas import tpu_sc as plsc

N = 50000
E = 800000
D = 65
EMB = 32
G = 1024

NP = 50176
NSEG = 2
QH = NP // NSEG
QHW = QH * D
DUMP = 1280
CW = QHW + DUMP
CFLAT = NSEG * QHW

ES = E // 16
K = 2000
NCH = ES // K
SL = NP // 16
HSL = SL // 2
ZB = 3920
TW = QHW // 16
NZC = TW // ZB

BLK = 512
NBLK = NP // BLK


def _rsqrt_newton(v):
    iv = lax.bitcast_convert_type(v, jnp.int32)
    y = lax.bitcast_convert_type(jnp.int32(0x5F3759DF) - (iv >> 1),
                                 jnp.float32)
    for _ in range(3):
        y = y * (1.5 - 0.5 * v * y * y)
    return y


_sc_mesh = plsc.VectorSubcoreMesh(core_axis_name="c", subcore_axis_name="s")


@functools.partial(
    pl.kernel,
    mesh=_sc_mesh,
    compiler_params=pltpu.CompilerParams(needs_layout_passes=False),
    out_type=[
        jax.ShapeDtypeStruct((NP,), jnp.float32),
        jax.ShapeDtypeStruct((CFLAT,), jnp.float32),
    ],
    scratch_types=[
        pltpu.VMEM((K,), jnp.float32),
        pltpu.VMEM((K,), jnp.int32),
        pltpu.VMEM((K,), jnp.int32),
        pltpu.VMEM((K,), jnp.int32),
        pltpu.VMEM((K,), jnp.float32),
        pltpu.VMEM((SL,), jnp.float32),
        pltpu.VMEM((ZB,), jnp.float32),
        pltpu.VMEM((ZB,), jnp.float32),
        pltpu.VMEM_SHARED((NP,), jnp.float32),
        pltpu.VMEM_SHARED((NP,), jnp.float32),
        pltpu.VMEM_SHARED((CW,), jnp.float32),
    ],
)
def _sc_scatter(x_hbm, src_hbm, dst_hbm, dinv_hbm, c_hbm,
                cxg, srcb, dstb, idxb, valb, onesb, zbuf, drainb,
                deg_sh, cx_sh, C_sh):
    c = lax.axis_index("c")
    s = lax.axis_index("s")

    def fill_ones(i, _):
        onesb[pl.ds(i * 16, 16)] = jnp.full((16,), 1.0, jnp.float32)
        return 0
    lax.fori_loop(0, SL // 16, fill_ones, 0)

    def fill_zero(i, _):
        zbuf[pl.ds(i * 16, 16)] = jnp.zeros((16,), jnp.float32)
        return 0
    lax.fori_loop(0, ZB // 16, fill_zero, 0)

    pltpu.sync_copy(onesb, deg_sh.at[pl.ds(s * SL, SL)])

    plsc.subcore_barrier()

    def deg_chunk(j, _):
        off = s * ES + j * K
        pltpu.sync_copy(dst_hbm.at[pl.ds(off, K)], dstb)
        pltpu.sync_copy(onesb.at[pl.ds(0, K)], deg_sh.at[dstb], add=True)
        return 0
    lax.fori_loop(0, NCH, deg_chunk, 0)

    plsc.subcore_barrier()

    pltpu.sync_copy(deg_sh.at[pl.ds(s * SL, SL)], onesb)

    def newton(g, _):
        v = onesb[pl.ds(g * 16, 16)]
        onesb[pl.ds(g * 16, 16)] = _rsqrt_newton(v)
        return 0
    lax.fori_loop(0, SL // 16, newton, 0)

    @pl.when(c == 0)
    def _():
        pltpu.sync_copy(onesb, dinv_hbm.at[pl.ds(s * SL, SL)])

    for h in range(2):
        pltpu.sync_copy(x_hbm.at[pl.ds(s * SL + h * HSL, HSL)],
                        srcb.at[pl.ds(0, HSL)])

        def pack(g, _):
            xi = srcb[pl.ds(g * 16, 16)]
            dv = onesb[pl.ds(h * HSL + g * 16, 16)]
            onesb[pl.ds(h * HSL + g * 16, 16)] = (
                2.0 * xi.astype(jnp.float32) + dv)
            return 0
        lax.fori_loop(0, HSL // 16, pack, 0)

    pltpu.sync_copy(onesb, cx_sh.at[pl.ds(s * SL, SL)])

    plsc.subcore_barrier()

    if True:
        q = c
        lo = q * QH

        def zero_c(k, _):
            pltpu.sync_copy(zbuf, C_sh.at[pl.ds(s * TW + k * ZB, ZB)])
            return 0
        lax.fori_loop(0, NZC, zero_c, 0)

        @pl.when(s == 0)
        def _():
            pltpu.sync_copy(zbuf.at[pl.ds(0, DUMP)],
                            C_sh.at[pl.ds(QHW, DUMP)])

        plsc.subcore_barrier()

        def edge_chunk(j, _):
            off = s * ES + j * K
            pltpu.sync_copy(src_hbm.at[pl.ds(off, K)], srcb)
            pltpu.sync_copy(dst_hbm.at[pl.ds(off, K)], dstb)
            pltpu.sync_copy(cx_sh.at[srcb], cxg)

            def group(g, _):
                dvec = dstb[pl.ds(g * 16, 16)]
                v = cxg[pl.ds(g * 16, 16)]
                xi = (v * 0.5).astype(jnp.int32)
                dval = v - 2.0 * xi.astype(jnp.float32)
                loc = dvec - lo
                inh = (loc >= 0) & (loc < QH)
                dmp = QHW + (dvec & 1023)
                idxb[pl.ds(g * 16, 16)] = jnp.where(inh, loc * D + xi, dmp)
                valb[pl.ds(g * 16, 16)] = dval
                return 0
            lax.fori_loop(0, K // 16, group, 0)

            pltpu.sync_copy(valb, C_sh.at[idxb], add=True)
            return 0
        lax.fori_loop(0, NCH, edge_chunk, 0)

        plsc.subcore_barrier()

        def drain(k, _):
            loc_off = s * TW + k * ZB
            pltpu.sync_copy(C_sh.at[pl.ds(loc_off, ZB)], drainb)
            pltpu.sync_copy(drainb, c_hbm.at[pl.ds(q * QHW + loc_off, ZB)])
            return 0
        lax.fori_loop(0, NZC, drain, 0)


def _tc_body(c_ref, dv_ref, x_ref, b_ref, wg_ref, bg_ref, wo_ref, out_ref,
             acc_ref):
    i = pl.program_id(0)
    cb = c_ref[...]
    dv = dv_ref[...]
    xb = x_ref[...]
    bb = b_ref[...]
    wgb = wg_ref[...].astype(jnp.bfloat16)
    ohx = jnp.where(
        lax.broadcasted_iota(jnp.int32, (BLK, D), 1) == xb, 1.0, 0.0)
    m = cb + dv * ohx
    mb = m.astype(jnp.bfloat16)
    mrb = (m - mb.astype(jnp.float32)).astype(jnp.bfloat16)
    dn = (((1,), (0,)), ((), ()))
    p = (lax.dot_general(mb, wgb, dn, preferred_element_type=jnp.float32)
         + lax.dot_general(mrb, wgb, dn, preferred_element_type=jnp.float32))
    h = jnp.tanh(dv * p + bg_ref[...])
    rows = lax.broadcasted_iota(jnp.int32, (BLK, 1), 0) + i * BLK
    h = jnp.where(rows < N, h, 0.0)
    ohb = (lax.broadcasted_iota(jnp.int32, (BLK, G), 1) == bb).astype(
        jnp.bfloat16)
    hb = h.astype(jnp.bfloat16)
    hrb = (h - hb.astype(jnp.float32)).astype(jnp.bfloat16)
    dp = (((0,), (0,)), ((), ()))
    contrib = (lax.dot_general(ohb, hb, dp,
                               preferred_element_type=jnp.float32)
               + lax.dot_general(ohb, hrb, dp,
                                 preferred_element_type=jnp.float32))

    @pl.when(i == 0)
    def _():
        acc_ref[...] = jnp.zeros((G, EMB), jnp.float32)

    acc_ref[...] += contrib

    @pl.when(i == NBLK - 1)
    def _():
        pb = acc_ref[...].astype(jnp.bfloat16)
        wob = wo_ref[...].astype(jnp.bfloat16)
        out_ref[...] = lax.dot_general(
            wob, pb, (((0,), (1,)), ((), ())),
            preferred_element_type=jnp.float32)


_tc_call = pl.pallas_call(
    _tc_body,
    grid=(NBLK,),
    in_specs=[
        pl.BlockSpec((BLK, D), lambda i: (i, 0)),
        pl.BlockSpec((BLK, 1), lambda i: (i, 0)),
        pl.BlockSpec((BLK, 1), lambda i: (i, 0)),
        pl.BlockSpec((BLK, 1), lambda i: (i, 0)),
        pl.BlockSpec((D, EMB), lambda i: (0, 0)),
        pl.BlockSpec((1, EMB), lambda i: (0, 0)),
        pl.BlockSpec((EMB, 1), lambda i: (0, 0)),
    ],
    out_specs=pl.BlockSpec((1, G), lambda i: (0, 0)),
    out_shape=jax.ShapeDtypeStruct((1, G), jnp.float32),
    scratch_shapes=[pltpu.VMEM((G, EMB), jnp.float32)],
)


def kernel(x, edge_index, batch_index, W_gcn, b_gcn, W_out, b_out):
    src = edge_index[0]
    dst = edge_index[1]
    pad = NP - N
    xpad = jnp.concatenate([x, jnp.zeros((pad,), jnp.int32)])
    dinv, cflat = _sc_scatter(xpad, src, dst)
    c2 = cflat.reshape(NP, D)
    b2 = jnp.concatenate(
        [batch_index, jnp.zeros((pad,), jnp.int32)]).reshape(NP, 1)
    acc = _tc_call(c2, dinv.reshape(NP, 1), xpad.reshape(NP, 1), b2,
                   W_gcn, b_gcn.reshape(1, EMB), W_out)
    return acc.reshape(G, 1) + b_out
